# baseline (speedup 1.0000x reference)
import jax
import jax.numpy as jnp
from jax.experimental import pallas as pl
from jax.experimental.pallas import tpu as pltpu

_BR = 64


def _flip_kernel(x_ref, sel_ref, o_ref):
    i = pl.program_id(0)
    rows = i * _BR + jax.lax.broadcasted_iota(jnp.int32, (_BR, 1), 0)
    hit = jnp.any(rows == sel_ref[...], axis=1, keepdims=True)
    sign = jnp.where(hit, -1.0, 1.0).astype(x_ref.dtype)
    o_ref[...] = x_ref[...] * sign


def kernel(data, selection):
    n, l = data.shape
    sel2d = selection.astype(jnp.int32).reshape(1, -1)
    return pl.pallas_call(
        _flip_kernel,
        grid=(n // _BR,),
        in_specs=[
            pl.BlockSpec((_BR, l), lambda i: (i, 0)),
            pl.BlockSpec(sel2d.shape, lambda i: (0, 0)),
        ],
        out_specs=pl.BlockSpec((_BR, l), lambda i: (i, 0)),
        out_shape=jax.ShapeDtypeStruct((n, l), data.dtype),
        compiler_params=pltpu.CompilerParams(
            dimension_semantics=("arbitrary",),
        ),
    )(data, sel2d)


# speedup vs baseline: 1.0204x; 1.0204x over previous
import jax
import jax.numpy as jnp
from jax.experimental import pallas as pl
from jax.experimental.pallas import tpu as pltpu

_BR = 192


def _flip_kernel(x_ref, sel_ref, o_ref):
    i = pl.program_id(0)
    rows = i * _BR + jax.lax.broadcasted_iota(jnp.int32, (_BR, 1), 0)
    hit = jnp.any(rows == sel_ref[...], axis=1, keepdims=True)
    sign = jnp.where(hit, -1.0, 1.0).astype(x_ref.dtype)
    o_ref[...] = x_ref[...] * sign


def kernel(data, selection):
    n, l = data.shape
    sel2d = selection.astype(jnp.int32).reshape(1, -1)
    return pl.pallas_call(
        _flip_kernel,
        grid=(pl.cdiv(n, _BR),),
        in_specs=[
            pl.BlockSpec((_BR, l), lambda i: (i, 0)),
            pl.BlockSpec(sel2d.shape, lambda i: (0, 0)),
        ],
        out_specs=pl.BlockSpec((_BR, l), lambda i: (i, 0)),
        out_shape=jax.ShapeDtypeStruct((n, l), data.dtype),
        compiler_params=pltpu.CompilerParams(
            dimension_semantics=("arbitrary",),
        ),
    )(data, sel2d)


# speedup vs baseline: 1.0217x; 1.0013x over previous
import jax
import jax.numpy as jnp
from jax.experimental import pallas as pl
from jax.experimental.pallas import tpu as pltpu

_BR = 224


def _flip_kernel(x_ref, sel_ref, o_ref):
    i = pl.program_id(0)
    rows = i * _BR + jax.lax.broadcasted_iota(jnp.int32, (_BR, 1), 0)
    hit = jnp.any(rows == sel_ref[...], axis=1, keepdims=True)
    sign = jnp.where(hit, -1.0, 1.0).astype(x_ref.dtype)
    o_ref[...] = x_ref[...] * sign


def kernel(data, selection):
    n, l = data.shape
    sel2d = selection.astype(jnp.int32).reshape(1, -1)
    return pl.pallas_call(
        _flip_kernel,
        grid=(pl.cdiv(n, _BR),),
        in_specs=[
            pl.BlockSpec((_BR, l), lambda i: (i, 0)),
            pl.BlockSpec(sel2d.shape, lambda i: (0, 0)),
        ],
        out_specs=pl.BlockSpec((_BR, l), lambda i: (i, 0)),
        out_shape=jax.ShapeDtypeStruct((n, l), data.dtype),
        compiler_params=pltpu.CompilerParams(
            dimension_semantics=("arbitrary",),
        ),
    )(data, sel2d)
